# baseline (speedup 1.0000x reference)
import functools

import jax
import jax.numpy as jnp
from jax import lax
from jax.experimental import pallas as pl
from jax.experimental.pallas import tpu as pltpu
from jax.experimental.pallas import tpu_sc as plsc

_N = 10000
_E = 320000
_D = 128
_H = 16
_C = 128

_NC = 2
_NS = 16
_CH = 80
_NROWS = _E // _CH
_TCH = _NROWS // (_NC * _NS)
_ACC_ROWS = 10240
_ZROWS = _ACC_ROWS // _NS


_PK = 8
_NP = _N // _PK
_ACC_P = _ACC_ROWS // _PK


def _tc1_body(x_ref, wl_ref, ws_ref, wm_ref, bl_ref, b1_ref, s_ref, m_ref):
    f32 = jnp.float32
    wl = wl_ref[...]
    ri = lax.broadcasted_iota(jnp.int32, (_PK * _D, _C), 0)
    ci = lax.broadcasted_iota(jnp.int32, (_PK * _D, _C), 1)
    mask = (ri // _D) == (ci // _H)
    tr = lax.broadcasted_iota(jnp.int32, (_PK * _D, _D), 0)
    tc = lax.broadcasted_iota(jnp.int32, (_PK * _D, _D), 1)
    Trow = jnp.where((tr % _D) == tc, 1.0, 0.0).astype(f32)
    sr = lax.broadcasted_iota(jnp.int32, (_H, _C), 0)
    sc = lax.broadcasted_iota(jnp.int32, (_H, _C), 1)
    Tcol = jnp.where((sc % _H) == sr, 1.0, 0.0).astype(f32)

    def pack_w(w1):
        wc = jnp.dot(wl, w1, preferred_element_type=f32)
        tiled = jnp.dot(Trow, jnp.dot(wc, Tcol, preferred_element_type=f32),
                        preferred_element_type=f32)
        return jnp.where(mask, tiled, 0.0)

    def pack_b(b1d):
        return jnp.dot(b1d, Tcol, preferred_element_type=f32)

    ws = ws_ref[...]
    wm = wm_ref[...]
    bl = bl_ref[...][None, :]
    b1 = b1_ref[...][None, :]
    cs = pack_b(jnp.dot(bl, ws, preferred_element_type=f32) + b1)
    cm = pack_b(jnp.dot(bl, wm, preferred_element_type=f32))
    xb = x_ref[...].reshape(_NP, _PK * _D)
    s_ref[...] = jnp.dot(xb, pack_w(ws), preferred_element_type=f32) + cs
    m_ref[...] = jnp.dot(xb, pack_w(wm), preferred_element_type=f32) + cm


def _tc1(x, W_lin, W1_self, W1_neigh, b_lin, b1):
    xp = x
    return pl.pallas_call(
        _tc1_body,
        out_shape=[
            jax.ShapeDtypeStruct((_NP, _C), jnp.float32),
            jax.ShapeDtypeStruct((_NP, _C), jnp.float32),
        ],
    )(xp, W_lin, W1_self, W1_neigh, b_lin, b1)


def _tc2_body(s_ref, p_ref, d_ref, h_ref):
    deg = d_ref[0, :_NP] + d_ref[1, :_NP]
    inv = 1.0 / jnp.maximum(deg, 1.0)
    h_ref[...] = jnp.tanh(s_ref[...] + (p_ref[0, :_NP] + p_ref[1, :_NP]) * inv)


def _tc2(s1, p1, d1):
    return pl.pallas_call(
        _tc2_body,
        out_shape=jax.ShapeDtypeStruct((_NP, _C), jnp.float32),
    )(s1, p1, d1)


def _tc3_body(h_ref, p_ref, d_ref, ws_ref, wn_ref, b2_ref, o_ref):
    f32 = jnp.float32
    deg = d_ref[0, :_NP] + d_ref[1, :_NP]
    inv = 1.0 / jnp.maximum(deg, 1.0)
    agg = (p_ref[0, :_NP] + p_ref[1, :_NP]) * inv
    h = h_ref[...]
    b2 = b2_ref[...][None, :]
    for k in range(_PK):
        sl = slice(k * _H, (k + 1) * _H)
        o_ref[:, k, :] = jnp.tanh(
            jnp.dot(h[:, sl], ws_ref[...], preferred_element_type=f32)
            + jnp.dot(agg[:, sl], wn_ref[...], preferred_element_type=f32)
            + b2
        )


def _tc3(h1, p2, d1, W2_self, W2_neigh, b2):
    out = pl.pallas_call(
        _tc3_body,
        out_shape=jax.ShapeDtypeStruct((_NP, _PK, _C), jnp.float32),
    )(h1, p2, d1, W2_self, W2_neigh, b2)
    return out.reshape(_N, _C)


_B = 4
_NB = 8


def _sc_agg_body(with_deg, m_hbm, ei_hbm, *rest):
    if with_deg:
        (out_hbm, deg_hbm, srcs, dsts, rows, ones_v, zero_v,
         acc, dacc, mcache, sem_g, sem_s, sem_d) = rest
    else:
        (out_hbm, srcs, dsts, rows, zero_v, acc, mcache,
         sem_g, sem_s) = rest
        deg_hbm = dacc = ones_v = sem_d = None

    c = lax.axis_index("c")
    s = lax.axis_index("s")

    zbase = pl.multiple_of(s * _ZROWS, 8)

    @pl.when(s < _NS - 1)
    def _():
        pltpu.async_copy(m_hbm.at[pl.ds(zbase, _ZROWS)],
                         mcache.at[pl.ds(zbase, _ZROWS)], sem_g)

    @pl.when(s == _NS - 1)
    def _():
        pltpu.async_copy(m_hbm.at[pl.ds(zbase, _N - (_NS - 1) * _ZROWS)],
                         mcache.at[pl.ds(zbase, _N - (_NS - 1) * _ZROWS)], sem_g)

    zero16 = jnp.zeros((16,), jnp.float32)

    @pl.loop(0, _ZROWS)
    def _(i):
        zero_v[i, :] = zero16

    pltpu.sync_copy(zero_v, acc.at[pl.ds(zbase, _ZROWS)])
    if with_deg:
        one16 = jnp.ones((16,), jnp.float32)

        @pl.loop(0, _CH)
        def _(i):
            ones_v[i, :] = one16

        pltpu.sync_copy(zero_v, dacc.at[pl.ds(zbase, _ZROWS)])

    @pl.when(s < _NS - 1)
    def _():
        pltpu.make_async_copy(m_hbm.at[pl.ds(zbase, _ZROWS)],
                              mcache.at[pl.ds(zbase, _ZROWS)], sem_g).wait()

    @pl.when(s == _NS - 1)
    def _():
        pltpu.make_async_copy(m_hbm.at[pl.ds(zbase, _N - (_NS - 1) * _ZROWS)],
                              mcache.at[pl.ds(zbase, _N - (_NS - 1) * _ZROWS)],
                              sem_g).wait()

    def g_issue(j, b):
        pltpu.async_copy(mcache.at[srcs.at[j]], rows.at[b], sem_g)

    def g_wait(j, b):
        pltpu.make_async_copy(mcache.at[srcs.at[j]], rows.at[b], sem_g).wait()

    def s_issue(j, b):
        pltpu.async_copy(rows.at[b], acc.at[dsts.at[j]], sem_s, add=True)
        if with_deg:
            pltpu.async_copy(ones_v, dacc.at[dsts.at[j]], sem_d, add=True)

    def s_wait(j, b):
        pltpu.make_async_copy(rows.at[b], acc.at[dsts.at[j]], sem_s).wait()
        if with_deg:
            pltpu.make_async_copy(ones_v, dacc.at[dsts.at[j]], sem_d).wait()

    def step(j, b, do_swait, do_gissue):
        g_wait(j, b)
        s_issue(j, b)
        if do_swait:
            s_wait(j - _B, (b - _B) % _NB)
        if do_gissue:
            g_issue(j + _B, (b + _B) % _NB)

    def run(nch):
        mid_hi = _NB * (nch // _NB - 1)
        for j in range(_B):
            g_issue(j, j % _NB)
        for j in range(_NB):
            step(j, j % _NB, j >= _B, True)

        @pl.loop(_NB, mid_hi, step=_NB)
        def _(j0):
            for b in range(_NB):
                step(j0 + b, b, True, True)

        for j in range(mid_hi, nch):
            step(j, j % _NB, True, j + _B < nch)
        for j in range(nch - _B, nch):
            s_wait(j, j % _NB)

    t = c * _NS + s
    rb = t * _TCH
    pltpu.sync_copy(ei_hbm.at[0, pl.ds(rb, _TCH)], srcs.at[pl.ds(0, _TCH)])
    pltpu.sync_copy(ei_hbm.at[1, pl.ds(rb, _TCH)], dsts.at[pl.ds(0, _TCH)])

    plsc.subcore_barrier()

    run(_TCH)

    plsc.subcore_barrier()
    pltpu.sync_copy(acc.at[pl.ds(zbase, _ZROWS)],
                    out_hbm.at[c, pl.ds(zbase, _ZROWS)])
    if with_deg:
        pltpu.sync_copy(dacc.at[pl.ds(zbase, _ZROWS)],
                        deg_hbm.at[c, pl.ds(zbase, _ZROWS)])


def _sc_agg(m, ei, with_deg):
    out_type = [jax.ShapeDtypeStruct((_NC, _ACC_ROWS, _H), jnp.float32)]
    scratch = [
        pltpu.VMEM((_TCH, _CH), jnp.int32),
        pltpu.VMEM((_TCH, _CH), jnp.int32),
        pltpu.VMEM((_NB, _CH, _H), jnp.float32),
        pltpu.VMEM((_ZROWS, _H), jnp.float32),
        pltpu.VMEM_SHARED((_ACC_ROWS, _H), jnp.float32),
        pltpu.VMEM_SHARED((_N, _H), jnp.float32),
        pltpu.SemaphoreType.DMA,
        pltpu.SemaphoreType.DMA,
    ]
    if with_deg:
        out_type.append(jax.ShapeDtypeStruct((_NC, _ACC_ROWS, _H), jnp.float32))
        scratch.insert(3, pltpu.VMEM((_CH, _H), jnp.float32))
        scratch.insert(5, pltpu.VMEM_SHARED((_ACC_ROWS, _H), jnp.float32))
        scratch.append(pltpu.SemaphoreType.DMA)
    mesh = plsc.VectorSubcoreMesh(core_axis_name="c", subcore_axis_name="s")
    return pl.kernel(
        functools.partial(_sc_agg_body, with_deg),
        out_type=out_type,
        mesh=mesh,
        scratch_types=scratch,
        compiler_params=pltpu.CompilerParams(use_tc_tiling_on_sc=False),
    )(m, ei)


def kernel(x, edge_index, W_lin, b_lin, W1_self, W1_neigh, b1,
           W2_self, W2_neigh, b2):
    ei = edge_index.reshape(2, _NROWS, _CH)

    s1, m1 = _tc1(x, W_lin, W1_self, W1_neigh, b_lin, b1)
    p1, d1 = _sc_agg(m1.reshape(_N, _H), ei, with_deg=True)
    p1 = p1.reshape(_NC, _ACC_P, _C)
    d1 = d1.reshape(_NC, _ACC_P, _C)
    h1 = _tc2(s1, p1, d1)
    (p2,) = _sc_agg(h1.reshape(_N, _H), ei, with_deg=False)
    p2 = p2.reshape(_NC, _ACC_P, _C)
    return _tc3(h1, p2, d1, W2_self, W2_neigh, b2)


# speedup vs baseline: 1.0024x; 1.0024x over previous
import functools

import jax
import jax.numpy as jnp
from jax import lax
from jax.experimental import pallas as pl
from jax.experimental.pallas import tpu as pltpu
from jax.experimental.pallas import tpu_sc as plsc

_N = 10000
_E = 320000
_D = 128
_H = 16
_C = 128

_NC = 2
_NS = 16
_CH = 80
_NROWS = _E // _CH
_TCH = _NROWS // (_NC * _NS)
_ACC_ROWS = 10240
_ZROWS = _ACC_ROWS // _NS


_PK = 8
_NP = _N // _PK
_ACC_P = _ACC_ROWS // _PK


def _tc1_body(x_ref, wl_ref, ws_ref, wm_ref, bl_ref, b1_ref, s_ref, m_ref):
    f32 = jnp.float32
    wl = wl_ref[...]
    ri = lax.broadcasted_iota(jnp.int32, (_PK * _D, _C), 0)
    ci = lax.broadcasted_iota(jnp.int32, (_PK * _D, _C), 1)
    mask = (ri // _D) == (ci // _H)
    tr = lax.broadcasted_iota(jnp.int32, (_PK * _D, _D), 0)
    tc = lax.broadcasted_iota(jnp.int32, (_PK * _D, _D), 1)
    Trow = jnp.where((tr % _D) == tc, 1.0, 0.0).astype(f32)
    sr = lax.broadcasted_iota(jnp.int32, (_H, _C), 0)
    sc = lax.broadcasted_iota(jnp.int32, (_H, _C), 1)
    Tcol = jnp.where((sc % _H) == sr, 1.0, 0.0).astype(f32)

    def pack_w(w1):
        wc = jnp.dot(wl, w1, preferred_element_type=f32)
        tiled = jnp.dot(Trow, jnp.dot(wc, Tcol, preferred_element_type=f32),
                        preferred_element_type=f32)
        return jnp.where(mask, tiled, 0.0)

    def pack_b(b1d):
        return jnp.dot(b1d, Tcol, preferred_element_type=f32)

    ws = ws_ref[...]
    wm = wm_ref[...]
    bl = bl_ref[...][None, :]
    b1 = b1_ref[...][None, :]
    cs = pack_b(jnp.dot(bl, ws, preferred_element_type=f32) + b1)
    cm = pack_b(jnp.dot(bl, wm, preferred_element_type=f32))
    xb = x_ref[...].reshape(_NP, _PK * _D)
    s_ref[...] = jnp.dot(xb, pack_w(ws), preferred_element_type=f32) + cs
    m_ref[...] = jnp.dot(xb, pack_w(wm), preferred_element_type=f32) + cm


def _tc1(x, W_lin, W1_self, W1_neigh, b_lin, b1):
    return pl.pallas_call(
        _tc1_body,
        out_shape=[
            jax.ShapeDtypeStruct((_NP, _C), jnp.float32),
            jax.ShapeDtypeStruct((_NP, _C), jnp.float32),
        ],
    )(x, W_lin, W1_self, W1_neigh, b_lin, b1)


def _tc2_body(s_ref, p_ref, d_ref, h_ref):
    deg = d_ref[0, :_NP] + d_ref[1, :_NP]
    inv = 1.0 / jnp.maximum(deg, 1.0)
    h_ref[...] = jnp.tanh(s_ref[...] + (p_ref[0, :_NP] + p_ref[1, :_NP]) * inv)


def _tc2(s1, p1, d1):
    return pl.pallas_call(
        _tc2_body,
        out_shape=jax.ShapeDtypeStruct((_NP, _C), jnp.float32),
    )(s1, p1, d1)


def _tc3_body(h_ref, p_ref, d_ref, ws_ref, wn_ref, b2_ref, o_ref):
    f32 = jnp.float32
    deg = d_ref[0, :_NP] + d_ref[1, :_NP]
    inv = 1.0 / jnp.maximum(deg, 1.0)
    agg = (p_ref[0, :_NP] + p_ref[1, :_NP]) * inv
    h = h_ref[...]
    b2 = b2_ref[...][None, :]
    for k in range(_PK):
        sl = slice(k * _H, (k + 1) * _H)
        o_ref[:, k, :] = jnp.tanh(
            jnp.dot(h[:, sl], ws_ref[...], preferred_element_type=f32)
            + jnp.dot(agg[:, sl], wn_ref[...], preferred_element_type=f32)
            + b2
        )


def _tc3(h1, p2, d1, W2_self, W2_neigh, b2):
    out = pl.pallas_call(
        _tc3_body,
        out_shape=jax.ShapeDtypeStruct((_NP, _PK, _C), jnp.float32),
    )(h1, p2, d1, W2_self, W2_neigh, b2)
    return out.reshape(_N, _C)


_B = 4
_NB = 8


def _sc_agg_body(with_deg, m_hbm, ei_hbm, *rest):
    if with_deg:
        (out_hbm, deg_hbm, srcs, dsts, rows, ones_v, zero_v,
         acc, dacc, mcache, sem_g, sem_s, sem_d) = rest
    else:
        (out_hbm, srcs, dsts, rows, zero_v, acc, mcache,
         sem_g, sem_s) = rest
        deg_hbm = dacc = ones_v = sem_d = None

    c = lax.axis_index("c")
    s = lax.axis_index("s")

    zbase = pl.multiple_of(s * _ZROWS, 8)

    @pl.when(s < _NS - 1)
    def _():
        pltpu.async_copy(m_hbm.at[pl.ds(zbase, _ZROWS)],
                         mcache.at[pl.ds(zbase, _ZROWS)], sem_g)

    @pl.when(s == _NS - 1)
    def _():
        pltpu.async_copy(m_hbm.at[pl.ds(zbase, _N - (_NS - 1) * _ZROWS)],
                         mcache.at[pl.ds(zbase, _N - (_NS - 1) * _ZROWS)], sem_g)

    zero16 = jnp.zeros((16,), jnp.float32)

    @pl.loop(0, _ZROWS)
    def _(i):
        zero_v[i, :] = zero16

    pltpu.sync_copy(zero_v, acc.at[pl.ds(zbase, _ZROWS)])
    if with_deg:
        one16 = jnp.ones((16,), jnp.float32)

        @pl.loop(0, _CH)
        def _(i):
            ones_v[i, :] = one16

        pltpu.sync_copy(zero_v, dacc.at[pl.ds(zbase, _ZROWS)])

    @pl.when(s < _NS - 1)
    def _():
        pltpu.make_async_copy(m_hbm.at[pl.ds(zbase, _ZROWS)],
                              mcache.at[pl.ds(zbase, _ZROWS)], sem_g).wait()

    @pl.when(s == _NS - 1)
    def _():
        pltpu.make_async_copy(m_hbm.at[pl.ds(zbase, _N - (_NS - 1) * _ZROWS)],
                              mcache.at[pl.ds(zbase, _N - (_NS - 1) * _ZROWS)],
                              sem_g).wait()

    def g_issue(j, b):
        pltpu.async_copy(mcache.at[srcs.at[j]], rows.at[b], sem_g)

    def g_wait(j, b):
        pltpu.make_async_copy(mcache.at[srcs.at[j]], rows.at[b], sem_g).wait()

    def s_issue(j, b):
        pltpu.async_copy(rows.at[b], acc.at[dsts.at[j]], sem_s, add=True)
        if with_deg:
            pltpu.async_copy(ones_v, dacc.at[dsts.at[j]], sem_d, add=True)

    def s_wait(j, b):
        pltpu.make_async_copy(rows.at[b], acc.at[dsts.at[j]], sem_s).wait()
        if with_deg:
            pltpu.make_async_copy(ones_v, dacc.at[dsts.at[j]], sem_d).wait()

    def step(j, b, do_swait, do_gissue):
        g_wait(j, b)
        s_issue(j, b)
        if do_swait:
            s_wait(j - _B, (b - _B) % _NB)
        if do_gissue:
            g_issue(j + _B, (b + _B) % _NB)

    def run(nch):
        mid_hi = _NB * (nch // _NB - 1)
        for j in range(_B):
            g_issue(j, j % _NB)
        for j in range(_NB):
            step(j, j % _NB, j >= _B, True)

        @pl.loop(_NB, mid_hi, step=_NB)
        def _(j0):
            for b in range(_NB):
                step(j0 + b, b, True, True)

        for j in range(mid_hi, nch):
            step(j, j % _NB, True, j + _B < nch)
        for j in range(nch - _B, nch):
            s_wait(j, j % _NB)

    t = c * _NS + s
    rb = t * _TCH
    pltpu.sync_copy(ei_hbm.at[0, pl.ds(rb, _TCH)], srcs.at[pl.ds(0, _TCH)])
    pltpu.sync_copy(ei_hbm.at[1, pl.ds(rb, _TCH)], dsts.at[pl.ds(0, _TCH)])

    plsc.subcore_barrier()

    run(_TCH)

    plsc.subcore_barrier()
    pltpu.sync_copy(acc.at[pl.ds(zbase, _ZROWS)],
                    out_hbm.at[c, pl.ds(zbase, _ZROWS)])
    if with_deg:
        pltpu.sync_copy(dacc.at[pl.ds(zbase, _ZROWS)],
                        deg_hbm.at[c, pl.ds(zbase, _ZROWS)])


def _sc_agg(m, ei, with_deg):
    out_type = [jax.ShapeDtypeStruct((_NC, _ACC_ROWS, _H), jnp.float32)]
    scratch = [
        pltpu.VMEM((_TCH, _CH), jnp.int32),
        pltpu.VMEM((_TCH, _CH), jnp.int32),
        pltpu.VMEM((_NB, _CH, _H), jnp.float32),
        pltpu.VMEM((_ZROWS, _H), jnp.float32),
        pltpu.VMEM_SHARED((_ACC_ROWS, _H), jnp.float32),
        pltpu.VMEM_SHARED((_N, _H), jnp.float32),
        pltpu.SemaphoreType.DMA,
        pltpu.SemaphoreType.DMA,
    ]
    if with_deg:
        out_type.append(jax.ShapeDtypeStruct((_NC, _ACC_ROWS, _H), jnp.float32))
        scratch.insert(3, pltpu.VMEM((_CH, _H), jnp.float32))
        scratch.insert(5, pltpu.VMEM_SHARED((_ACC_ROWS, _H), jnp.float32))
        scratch.append(pltpu.SemaphoreType.DMA)
    mesh = plsc.VectorSubcoreMesh(core_axis_name="c", subcore_axis_name="s")
    return pl.kernel(
        functools.partial(_sc_agg_body, with_deg),
        out_type=out_type,
        mesh=mesh,
        scratch_types=scratch,
        compiler_params=pltpu.CompilerParams(use_tc_tiling_on_sc=False),
    )(m, ei)


def kernel(x, edge_index, W_lin, b_lin, W1_self, W1_neigh, b1,
           W2_self, W2_neigh, b2):
    ei = edge_index.reshape(2, _NROWS, _CH)

    s1, m1 = _tc1(x, W_lin, W1_self, W1_neigh, b_lin, b1)
    p1, d1 = _sc_agg(m1.reshape(_N, _H), ei, with_deg=True)
    p1 = p1.reshape(_NC, _ACC_P, _C)
    d1 = d1.reshape(_NC, _ACC_P, _C)
    h1 = _tc2(s1, p1, d1)
    (p2,) = _sc_agg(h1.reshape(_N, _H), ei, with_deg=False)
    p2 = p2.reshape(_NC, _ACC_P, _C)
    return _tc3(h1, p2, d1, W2_self, W2_neigh, b2)
